# Initial kernel scaffold; baseline (speedup 1.0000x reference)
import jax
import jax.numpy as jnp
from jax import lax
from jax.experimental import pallas as pl
from jax.experimental.pallas import tpu as pltpu
from jax.experimental.pallas import tpu_sc as plsc

VOCAB = 1000000
EMB_DIM = 32
BATCH = 16384
FIELDS = 26

NC = 2
NS = 16
L = 16
NW = NC * NS

TOTAL = BATCH * FIELDS
PER_W = TOTAL // NW
CHUNK = 128
NCHUNK = PER_W // CHUNK
WORDS = EMB_DIM // 4
GROUPS = CHUNK * EMB_DIM // (4 * L)


def _body(qw_hbm, idx_hbm, sb_hbm, out_hbm, idx_v, rows_v, out_v, sb_v):
    wid = lax.axis_index("s") * NC + lax.axis_index("c")

    pltpu.sync_copy(idx_hbm.at[wid], idx_v)
    pltpu.sync_copy(sb_hbm, sb_v)

    vs = sb_v[0, :]
    vb = sb_v[1, :]
    i16 = lax.broadcasted_iota(jnp.int32, (L,), 0)
    row_off = i16 >> 3
    col_idx = i16 & 7
    out_offs = [4 * i16 + j for j in range(4)]

    def chunk_body(c, carry):
        pltpu.sync_copy(qw_hbm.at[idx_v.at[c]], rows_v)

        def group_body(g, carry2):
            w = plsc.load_gather(rows_v, [2 * g + row_off, col_idx])
            obase = (4 * L) * g
            for j in range(4):
                t = (w << (24 - 8 * j)) >> 24 if j < 3 else (w >> 24)
                y = (t.astype(jnp.float32) - vb) * vs
                plsc.store_scatter(out_v, [obase + out_offs[j]], y)
            return carry2

        lax.fori_loop(0, GROUPS, group_body, 0, unroll=2)

        off = (wid * NCHUNK + c) * (CHUNK * EMB_DIM)
        pltpu.sync_copy(out_v, out_hbm.at[pl.ds(off, CHUNK * EMB_DIM)])
        return carry

    lax.fori_loop(0, NCHUNK, chunk_body, 0)


@jax.jit
def kernel(x, qweight, scale, bias):
    qw32 = lax.bitcast_convert_type(qweight.reshape(VOCAB, WORDS, 4), jnp.int32)
    idx = x.reshape(-1).astype(jnp.int32).reshape(NW, NCHUNK, CHUNK)
    sb = jnp.stack([jnp.full((L,), scale, jnp.float32),
                    jnp.full((L,), bias.astype(jnp.float32))])

    mesh = plsc.VectorSubcoreMesh(core_axis_name="c", subcore_axis_name="s",
                                  num_cores=NC, num_subcores=NS)
    out = pl.kernel(
        _body,
        out_type=jax.ShapeDtypeStruct((TOTAL * EMB_DIM,), jnp.float32),
        mesh=mesh,
        scratch_types=[
            pltpu.VMEM((NCHUNK, CHUNK), jnp.int32),
            pltpu.VMEM((CHUNK, WORDS), jnp.int32),
            pltpu.VMEM((CHUNK * EMB_DIM,), jnp.float32),
            pltpu.VMEM((2, L), jnp.float32),
        ],
    )(qw32, idx, sb)
    return out.reshape(BATCH, FIELDS, EMB_DIM)


# speedup vs baseline: 5.4103x; 5.4103x over previous
import jax
import jax.numpy as jnp
from jax import lax
from jax.experimental import pallas as pl
from jax.experimental.pallas import tpu as pltpu
from jax.experimental.pallas import tpu_sc as plsc

VOCAB = 1000000
EMB_DIM = 32
BATCH = 16384
FIELDS = 26

NC = 2
NS = 16
L = 16
NW = NC * NS

TOTAL = BATCH * FIELDS
PER_W = TOTAL // NW
CHUNK = 128
NCHUNK = PER_W // CHUNK
WORDS = EMB_DIM // 4
GROUPS = CHUNK * EMB_DIM // (4 * L)


def _body(qw_hbm, idx_hbm, sb_hbm, out_hbm, idx_v, rows_v, out_v, sb_v):
    wid = lax.axis_index("s") * NC + lax.axis_index("c")

    pltpu.sync_copy(idx_hbm.at[wid], idx_v)
    pltpu.sync_copy(sb_hbm, sb_v)

    vs = sb_v[0, :]
    vb = sb_v[1, :]
    i16 = lax.broadcasted_iota(jnp.int32, (L,), 0)
    row_off = i16 >> 3
    col_idx = i16 & 7
    out_offs = [4 * i16 + j for j in range(4)]

    def chunk_body(c, carry):
        pltpu.sync_copy(qw_hbm.at[idx_v.at[c]], rows_v)

        def group_body(g, carry2):
            w = plsc.load_gather(rows_v, [2 * g + row_off, col_idx])
            obase = (4 * L) * g
            for j in range(4):
                t = (w << (24 - 8 * j)) >> 24 if j < 3 else (w >> 24)
                y = (t.astype(jnp.float32) - vb) * vs
                plsc.store_scatter(out_v, [obase + out_offs[j]], y)
            return carry2

        lax.fori_loop(0, GROUPS, group_body, 0, unroll=2)

        off = (wid * NCHUNK + c) * (CHUNK * EMB_DIM)
        pltpu.sync_copy(out_v, out_hbm.at[pl.ds(off, CHUNK * EMB_DIM)])
        return carry

    lax.fori_loop(0, NCHUNK, chunk_body, 0)


@jax.jit
def kernel(x, qweight, scale, bias):
    qw32 = lax.bitcast_convert_type(qweight.reshape(VOCAB, WORDS, 4), jnp.int32)
    idx = x.reshape(-1).astype(jnp.int32).reshape(NW, NCHUNK, CHUNK)
    sb = jnp.stack([jnp.full((L,), scale, jnp.float32),
                    jnp.full((L,), bias.astype(jnp.float32))])

    mesh = plsc.VectorSubcoreMesh(core_axis_name="c", subcore_axis_name="s",
                                  num_cores=NC, num_subcores=NS)
    out = pl.kernel(
        _body,
        out_type=jax.ShapeDtypeStruct((TOTAL * EMB_DIM,), jnp.float32),
        mesh=mesh,
        compiler_params=pltpu.CompilerParams(needs_layout_passes=False,
                                             use_tc_tiling_on_sc=False),
        scratch_types=[
            pltpu.VMEM((NCHUNK, CHUNK), jnp.int32),
            pltpu.VMEM((CHUNK, WORDS), jnp.int32),
            pltpu.VMEM((CHUNK * EMB_DIM,), jnp.float32),
            pltpu.VMEM((2, L), jnp.float32),
        ],
    )(qw32, idx, sb)
    return out.reshape(BATCH, FIELDS, EMB_DIM)


# speedup vs baseline: 5.7489x; 1.0626x over previous
import jax
import jax.numpy as jnp
from jax import lax
from jax.experimental import pallas as pl
from jax.experimental.pallas import tpu as pltpu
from jax.experimental.pallas import tpu_sc as plsc

VOCAB = 1000000
EMB_DIM = 32
BATCH = 16384
FIELDS = 26

NC = 2
NS = 16
L = 16
NW = NC * NS

TOTAL = BATCH * FIELDS
PER_W = TOTAL // NW
CHUNK = 128
NCHUNK = PER_W // CHUNK
WORDS = EMB_DIM // 4
GROUPS = CHUNK * EMB_DIM // (4 * L)
CELEMS = CHUNK * EMB_DIM


def _body(qw_hbm, idx_hbm, sb_hbm, out_hbm, idx_v,
          rows0, rows1, out0, out1, sb_v, gs0, gs1, ss0, ss1):
    wid = lax.axis_index("s") * NC + lax.axis_index("c")

    pltpu.sync_copy(idx_hbm.at[wid], idx_v)
    pltpu.sync_copy(sb_hbm, sb_v)

    rows = (rows0, rows1)
    outs = (out0, out1)
    gsem = (gs0, gs1)
    ssem = (ss0, ss1)

    vs = sb_v[0, :]
    vb = sb_v[1, :]
    i16 = lax.broadcasted_iota(jnp.int32, (L,), 0)
    row_off = i16 >> 3
    col_idx = i16 & 7
    out_offs = [4 * i16 + j for j in range(4)]

    obase_hbm = wid * (NCHUNK * CELEMS)

    pltpu.async_copy(qw_hbm.at[idx_v.at[0]], rows0, gs0)

    def pair_body(c2, carry):
        for b in range(2):
            c = 2 * c2 + b
            pltpu.make_async_copy(qw_hbm.at[idx_v.at[c]], rows[b],
                                  gsem[b]).wait()

            @pl.when(c + 1 < NCHUNK)
            def _():
                pltpu.async_copy(qw_hbm.at[idx_v.at[c + 1]], rows[1 - b],
                                 gsem[1 - b])

            @pl.when(c >= 2)
            def _():
                pltpu.make_async_copy(
                    outs[b],
                    out_hbm.at[pl.ds(obase_hbm + (c - 2) * CELEMS, CELEMS)],
                    ssem[b]).wait()

            rbuf = rows[b]
            obuf = outs[b]

            def group_body(g, carry2):
                w = plsc.load_gather(rbuf, [2 * g + row_off, col_idx])
                og = (4 * L) * g
                for j in range(4):
                    t = (w << (24 - 8 * j)) >> 24 if j < 3 else (w >> 24)
                    y = (t.astype(jnp.float32) - vb) * vs
                    plsc.store_scatter(obuf, [og + out_offs[j]], y)
                return carry2

            lax.fori_loop(0, GROUPS, group_body, 0, unroll=2)

            pltpu.async_copy(obuf,
                             out_hbm.at[pl.ds(obase_hbm + c * CELEMS, CELEMS)],
                             ssem[b])
        return carry

    lax.fori_loop(0, NCHUNK // 2, pair_body, 0)

    for b in range(2):
        c = NCHUNK - 2 + b
        pltpu.make_async_copy(outs[b],
                              out_hbm.at[pl.ds(obase_hbm + c * CELEMS, CELEMS)],
                              ssem[b]).wait()


@jax.jit
def kernel(x, qweight, scale, bias):
    qw32 = lax.bitcast_convert_type(qweight.reshape(VOCAB, WORDS, 4), jnp.int32)
    idx = x.reshape(-1).astype(jnp.int32).reshape(NW, NCHUNK, CHUNK)
    sb = jnp.stack([jnp.full((L,), scale, jnp.float32),
                    jnp.full((L,), bias.astype(jnp.float32))])

    mesh = plsc.VectorSubcoreMesh(core_axis_name="c", subcore_axis_name="s",
                                  num_cores=NC, num_subcores=NS)
    out = pl.kernel(
        _body,
        out_type=jax.ShapeDtypeStruct((TOTAL * EMB_DIM,), jnp.float32),
        mesh=mesh,
        compiler_params=pltpu.CompilerParams(needs_layout_passes=False,
                                             use_tc_tiling_on_sc=False),
        scratch_types=[
            pltpu.VMEM((NCHUNK, CHUNK), jnp.int32),
            pltpu.VMEM((CHUNK, WORDS), jnp.int32),
            pltpu.VMEM((CHUNK, WORDS), jnp.int32),
            pltpu.VMEM((CELEMS,), jnp.float32),
            pltpu.VMEM((CELEMS,), jnp.float32),
            pltpu.VMEM((2, L), jnp.float32),
            pltpu.SemaphoreType.DMA,
            pltpu.SemaphoreType.DMA,
            pltpu.SemaphoreType.DMA,
            pltpu.SemaphoreType.DMA,
        ],
    )(qw32, idx, sb)
    return out.reshape(BATCH, FIELDS, EMB_DIM)


# speedup vs baseline: 6.8284x; 1.1878x over previous
import jax
import jax.numpy as jnp
from jax import lax
from jax.experimental import pallas as pl
from jax.experimental.pallas import tpu as pltpu
from jax.experimental.pallas import tpu_sc as plsc

VOCAB = 1000000
EMB_DIM = 32
BATCH = 16384
FIELDS = 26

NC = 2
NS = 16
L = 16
NW = NC * NS

TOTAL = BATCH * FIELDS
PER_W = TOTAL // NW
CHUNK = 128
NCHUNK = PER_W // CHUNK
NBUF = 4


def _body(qw_hbm, idx_hbm, out_hbm, idx_v,
          rows0, rows1, rows2, rows3,
          gs0, gs1, gs2, gs3, ss0, ss1, ss2, ss3):
    wid = lax.axis_index("s") * NC + lax.axis_index("c")

    pltpu.sync_copy(idx_hbm.at[wid], idx_v)

    rows = (rows0, rows1, rows2, rows3)
    gsem = (gs0, gs1, gs2, gs3)
    ssem = (ss0, ss1, ss2, ss3)

    rbase = wid * PER_W

    def fire(c, b):
        pltpu.async_copy(qw_hbm.at[idx_v.at[c]], rows[b], gsem[b])

    def gwait(c, b):
        pltpu.make_async_copy(qw_hbm.at[idx_v.at[c]], rows[b], gsem[b]).wait()

    def sfire(c, b):
        pltpu.async_copy(rows[b],
                         out_hbm.at[pl.ds(rbase + c * CHUNK, CHUNK)], ssem[b])

    def swait(c, b):
        pltpu.make_async_copy(rows[b],
                              out_hbm.at[pl.ds(rbase + c * CHUNK, CHUNK)],
                              ssem[b]).wait()

    for b in range(NBUF):
        fire(b, b)

    def rot_body(q, carry):
        for b in range(NBUF):
            c = NBUF * q + b
            gwait(c, b)
            sfire(c, b)

            @pl.when(c + NBUF < NCHUNK)
            def _():
                swait(c, b)
                fire(c + NBUF, b)
        return carry

    lax.fori_loop(0, NCHUNK // NBUF, rot_body, 0)

    for b in range(NBUF):
        c = NCHUNK - NBUF + b
        swait(c, b)


@jax.jit
def kernel(x, qweight, scale, bias):
    idx = x.reshape(-1).astype(jnp.int32).reshape(NW, NCHUNK, CHUNK)

    mesh = plsc.VectorSubcoreMesh(core_axis_name="c", subcore_axis_name="s",
                                  num_cores=NC, num_subcores=NS)
    g = pl.kernel(
        _body,
        out_type=jax.ShapeDtypeStruct((TOTAL, EMB_DIM), jnp.int8),
        mesh=mesh,
        compiler_params=pltpu.CompilerParams(needs_layout_passes=False,
                                             use_tc_tiling_on_sc=False),
        scratch_types=[
            pltpu.VMEM((NCHUNK, CHUNK), jnp.int32),
            pltpu.VMEM((CHUNK, EMB_DIM), jnp.int8),
            pltpu.VMEM((CHUNK, EMB_DIM), jnp.int8),
            pltpu.VMEM((CHUNK, EMB_DIM), jnp.int8),
            pltpu.VMEM((CHUNK, EMB_DIM), jnp.int8),
            pltpu.SemaphoreType.DMA,
            pltpu.SemaphoreType.DMA,
            pltpu.SemaphoreType.DMA,
            pltpu.SemaphoreType.DMA,
            pltpu.SemaphoreType.DMA,
            pltpu.SemaphoreType.DMA,
            pltpu.SemaphoreType.DMA,
            pltpu.SemaphoreType.DMA,
        ],
    )(qweight, idx)
    res = g.reshape(BATCH, FIELDS, EMB_DIM).astype(jnp.float32)
    return (res - bias.astype(jnp.float32)) * scale


# speedup vs baseline: 9.9142x; 1.4519x over previous
import jax
import jax.numpy as jnp
from jax import lax
from jax.experimental import pallas as pl
from jax.experimental.pallas import tpu as pltpu
from jax.experimental.pallas import tpu_sc as plsc

VOCAB = 1000000
EMB_DIM = 32
BATCH = 16384
FIELDS = 26

NC = 2
NS = 16
L = 16
NW = NC * NS

TOTAL = BATCH * FIELDS
PER_W = TOTAL // NW
CHUNK = 128
NCHUNK = PER_W // CHUNK
NBUF = 4


def _body(qw_hbm, idx_hbm, out_hbm, idx_v,
          rows0, rows1, rows2, rows3,
          gs0, gs1, gs2, gs3, ss0, ss1, ss2, ss3):
    wid = lax.axis_index("s") * NC + lax.axis_index("c")

    pltpu.sync_copy(idx_hbm.at[wid], idx_v)

    rows = (rows0, rows1, rows2, rows3)
    gsem = (gs0, gs1, gs2, gs3)
    ssem = (ss0, ss1, ss2, ss3)

    rbase = wid * PER_W

    def fire(c, b):
        pltpu.async_copy(qw_hbm.at[idx_v.at[c]], rows[b], gsem[b])

    def gwait(c, b):
        pltpu.make_async_copy(qw_hbm.at[idx_v.at[c]], rows[b], gsem[b]).wait()

    def sfire(c, b):
        pltpu.async_copy(rows[b],
                         out_hbm.at[pl.ds(rbase + c * CHUNK, CHUNK)], ssem[b])

    def swait(c, b):
        pltpu.make_async_copy(rows[b],
                              out_hbm.at[pl.ds(rbase + c * CHUNK, CHUNK)],
                              ssem[b]).wait()

    for b in range(NBUF):
        fire(b, b)

    def rot_body(q, carry):
        for b in range(NBUF):
            c = NBUF * q + b
            gwait(c, b)
            sfire(c, b)

            @pl.when(c + NBUF < NCHUNK)
            def _():
                swait(c, b)
                fire(c + NBUF, b)
        return carry

    lax.fori_loop(0, NCHUNK // NBUF, rot_body, 0)

    for b in range(NBUF):
        c = NCHUNK - NBUF + b
        swait(c, b)


def _dequant_body(g_ref, sb_ref, o_ref):
    v = g_ref[...].astype(jnp.float32)
    y = (v - sb_ref[1]) * sb_ref[0]
    for f in range(FIELDS):
        o_ref[f] = y[:, f * EMB_DIM:(f + 1) * EMB_DIM].T


@jax.jit
def kernel(x, qweight, scale, bias):
    idx = x.reshape(-1).astype(jnp.int32).reshape(NW, NCHUNK, CHUNK)

    mesh = plsc.VectorSubcoreMesh(core_axis_name="c", subcore_axis_name="s",
                                  num_cores=NC, num_subcores=NS)
    g = pl.kernel(
        _body,
        out_type=jax.ShapeDtypeStruct((TOTAL, EMB_DIM), jnp.int8),
        mesh=mesh,
        compiler_params=pltpu.CompilerParams(needs_layout_passes=False,
                                             use_tc_tiling_on_sc=False),
        scratch_types=[
            pltpu.VMEM((NCHUNK, CHUNK), jnp.int32),
            pltpu.VMEM((CHUNK, EMB_DIM), jnp.int8),
            pltpu.VMEM((CHUNK, EMB_DIM), jnp.int8),
            pltpu.VMEM((CHUNK, EMB_DIM), jnp.int8),
            pltpu.VMEM((CHUNK, EMB_DIM), jnp.int8),
            pltpu.SemaphoreType.DMA,
            pltpu.SemaphoreType.DMA,
            pltpu.SemaphoreType.DMA,
            pltpu.SemaphoreType.DMA,
            pltpu.SemaphoreType.DMA,
            pltpu.SemaphoreType.DMA,
            pltpu.SemaphoreType.DMA,
            pltpu.SemaphoreType.DMA,
        ],
    )(qweight, idx)

    g2 = g.reshape(BATCH, FIELDS * EMB_DIM)
    sb = jnp.stack([scale, bias.astype(jnp.float32)])
    BB = 512
    out_t = pl.pallas_call(
        _dequant_body,
        grid=(BATCH // BB,),
        in_specs=[
            pl.BlockSpec((BB, FIELDS * EMB_DIM), lambda j: (j, 0)),
            pl.BlockSpec(memory_space=pltpu.SMEM),
        ],
        out_specs=pl.BlockSpec((FIELDS, EMB_DIM, BB), lambda j: (0, 0, j)),
        out_shape=jax.ShapeDtypeStruct((FIELDS, EMB_DIM, BATCH), jnp.float32),
    )(g2, sb)
    return out_t.transpose(2, 0, 1)
